# baseline (speedup 1.0000x reference)
import functools

import jax
import jax.numpy as jnp
from jax.experimental import pallas as pl
from jax.experimental.pallas import tpu as pltpu

F32 = jnp.float32
BF16 = jnp.bfloat16


def _embed_body(x_ref, t_ref, xW_ref, xb_ref, tW_ref, tb_ref, vx_ref, vt_ref):
    vx_ref[...] = (
        jnp.dot(x_ref[...], xW_ref[...], preferred_element_type=F32) + xb_ref[...]
    )
    vt_ref[...] = (
        jnp.dot(t_ref[...], tW_ref[...], preferred_element_type=F32) + tb_ref[...]
    )


def _embed_v(x, t, xWt, xb, tWt, tb, bm):
    n = x.shape[0]
    nm = n // bm
    vx, vt = pl.pallas_call(
        _embed_body,
        grid=(nm,),
        in_specs=[
            pl.BlockSpec((bm, x.shape[1]), lambda m: (m, 0)),
            pl.BlockSpec((bm, t.shape[1]), lambda m: (m, 0)),
            pl.BlockSpec(xWt.shape, lambda m: (0, 0)),
            pl.BlockSpec(xb.shape, lambda m: (0, 0)),
            pl.BlockSpec(tWt.shape, lambda m: (0, 0)),
            pl.BlockSpec(tb.shape, lambda m: (0, 0)),
        ],
        out_specs=[
            pl.BlockSpec((bm, xWt.shape[1]), lambda m: (m, 0)),
            pl.BlockSpec((bm, tWt.shape[1]), lambda m: (m, 0)),
        ],
        out_shape=[
            jax.ShapeDtypeStruct((n, xWt.shape[1]), F32),
            jax.ShapeDtypeStruct((n, tWt.shape[1]), F32),
        ],
    )(x, t, xWt, xb, tWt, tb)
    return vx, vt


def _stage_body(nk, e_ref, r_ref, s_ref, Wa_ref, Wb_ref, b1_ref, W2_ref, b2_ref,
                out_ref, acc_ref):
    k = pl.program_id(1)
    part = jnp.dot(e_ref[...].astype(BF16), r_ref[...], preferred_element_type=F32)

    @pl.when(k == 0)
    def _():
        acc_ref[...] = part

    @pl.when(k > 0)
    def _():
        acc_ref[...] += part

    @pl.when(k == nk - 1)
    def _():
        h = (
            jnp.dot(s_ref[...], Wa_ref[...], preferred_element_type=F32)
            + jnp.dot(acc_ref[...], Wb_ref[...], preferred_element_type=F32)
            + b1_ref[...]
        )
        h = jnp.maximum(h, 0.0)
        o = jnp.dot(h, W2_ref[...], preferred_element_type=F32) + b2_ref[...]
        out_ref[...] = o.astype(out_ref.dtype)


def _stage(e, r_bf, s, Wa, Wb, b1, W2, b2, out_dtype, bm, bk):
    M, K = e.shape
    N = r_bf.shape[1]
    H = Wa.shape[1]
    ds = s.shape[1]
    No = W2.shape[1]
    nm, nk = M // bm, K // bk
    return pl.pallas_call(
        functools.partial(_stage_body, nk),
        grid=(nm, nk),
        in_specs=[
            pl.BlockSpec((bm, bk), lambda m, k: (m, k)),
            pl.BlockSpec((bk, N), lambda m, k: (k, 0)),
            pl.BlockSpec((bm, ds), lambda m, k: (m, 0)),
            pl.BlockSpec((ds, H), lambda m, k: (0, 0)),
            pl.BlockSpec((N, H), lambda m, k: (0, 0)),
            pl.BlockSpec((1, H), lambda m, k: (0, 0)),
            pl.BlockSpec((H, No), lambda m, k: (0, 0)),
            pl.BlockSpec((1, No), lambda m, k: (0, 0)),
        ],
        out_specs=pl.BlockSpec((bm, No), lambda m, k: (m, 0)),
        out_shape=jax.ShapeDtypeStruct((M, No), out_dtype),
        scratch_shapes=[pltpu.VMEM((bm, N), F32)],
        compiler_params=pltpu.CompilerParams(
            dimension_semantics=("parallel", "arbitrary")
        ),
    )(e, r_bf, s, Wa, Wb, b1, W2, b2)


def kernel(c, x, t, k_f, e_cv, e_vc, e_v_veh, cW, cb, xW, xb, tW, tb, kW, kb,
           f1W, f1b, f2W, f2b, f3W, f3b, f4W, f4b, f5W, f5b, f6W, f6b):
    emb = cW.shape[0]

    W1 = f1W.T
    W1a, W1b = W1[:emb], W1[emb:]
    W_c1 = cW.T @ W1a
    b1f = (cb @ W1a + f1b)[None, :]
    W2 = f2W.T
    b2 = f2b[None, :]

    W3 = f3W.T
    W3a, W3b = W3[:emb], W3[emb:]
    b3 = f3b[None, :]
    W4 = f4W.T
    b4 = f4b[None, :]

    W5 = f5W.T
    W5a, W5b = W5[:emb], W5[emb:]
    W_k5 = kW.T @ W5b
    b5f = (kb @ W5b + f5b)[None, :]
    W6 = f6W.T
    b6 = f6b[None, :]

    bm, bk = 512, 1024

    vx, vt = _embed_v(x, t, xW.T, xb[None, :], tW.T, tb[None, :], bm=1024)
    v = jnp.concatenate([vx, vt], axis=0)
    v_bf = v.astype(BF16)

    cc_bf = _stage(e_cv, v_bf, c, W_c1, W1b, b1f, W2, b2, BF16, bm, bk)
    vv_bf = _stage(e_vc, cc_bf, v, W3a, W3b, b3, W4, b4, BF16, bm, bk)
    out = _stage(e_v_veh, vv_bf, k_f, W_k5, W5a, b5f, W6, b6, F32, bm, bk)
    return out


# speedup vs baseline: 1.4514x; 1.4514x over previous
import jax
import jax.numpy as jnp
from jax.experimental import pallas as pl
from jax.experimental.pallas import tpu as pltpu

F32 = jnp.float32


def _embed_body(x_ref, t_ref, xW_ref, xb_ref, tW_ref, tb_ref, vx_ref, vt_ref):
    vx_ref[...] = (
        jnp.dot(x_ref[...], xW_ref[...], preferred_element_type=F32) + xb_ref[...]
    )
    vt_ref[...] = (
        jnp.dot(t_ref[...], tW_ref[...], preferred_element_type=F32) + tb_ref[...]
    )


def _embed_v(x, t, xWt, xb, tWt, tb, bm):
    n = x.shape[0]
    nm = n // bm
    vx, vt = pl.pallas_call(
        _embed_body,
        grid=(nm,),
        in_specs=[
            pl.BlockSpec((bm, x.shape[1]), lambda m: (m, 0)),
            pl.BlockSpec((bm, t.shape[1]), lambda m: (m, 0)),
            pl.BlockSpec(xWt.shape, lambda m: (0, 0)),
            pl.BlockSpec(xb.shape, lambda m: (0, 0)),
            pl.BlockSpec(tWt.shape, lambda m: (0, 0)),
            pl.BlockSpec(tb.shape, lambda m: (0, 0)),
        ],
        out_specs=[
            pl.BlockSpec((bm, xWt.shape[1]), lambda m: (m, 0)),
            pl.BlockSpec((bm, tWt.shape[1]), lambda m: (m, 0)),
        ],
        out_shape=[
            jax.ShapeDtypeStruct((n, xWt.shape[1]), F32),
            jax.ShapeDtypeStruct((n, tWt.shape[1]), F32),
        ],
    )(x, t, xWt, xb, tWt, tb)
    return vx, vt


def _stage_body(e_ref, r_ref, s_ref, Wa_ref, Wb_ref, b1_ref, W2_ref, b2_ref,
                out_ref):
    agg = jnp.dot(e_ref[...], r_ref[...], preferred_element_type=F32)
    h = (
        jnp.dot(s_ref[...], Wa_ref[...], preferred_element_type=F32)
        + jnp.dot(agg, Wb_ref[...], preferred_element_type=F32)
        + b1_ref[...]
    )
    h = jnp.maximum(h, 0.0)
    o = jnp.dot(h, W2_ref[...], preferred_element_type=F32) + b2_ref[...]
    out_ref[...] = o


def _stage(e, r, s, Wa, Wb, b1, W2, b2, bm):
    M, K = e.shape
    N = r.shape[1]
    H = Wa.shape[1]
    ds = s.shape[1]
    No = W2.shape[1]
    nm = M // bm
    return pl.pallas_call(
        _stage_body,
        grid=(nm,),
        in_specs=[
            pl.BlockSpec((bm, K), lambda m: (m, 0)),
            pl.BlockSpec((K, N), lambda m: (0, 0)),
            pl.BlockSpec((bm, ds), lambda m: (m, 0)),
            pl.BlockSpec((ds, H), lambda m: (0, 0)),
            pl.BlockSpec((N, H), lambda m: (0, 0)),
            pl.BlockSpec((1, H), lambda m: (0, 0)),
            pl.BlockSpec((H, No), lambda m: (0, 0)),
            pl.BlockSpec((1, No), lambda m: (0, 0)),
        ],
        out_specs=pl.BlockSpec((bm, No), lambda m: (m, 0)),
        out_shape=jax.ShapeDtypeStruct((M, No), F32),
        compiler_params=pltpu.CompilerParams(
            dimension_semantics=("arbitrary",)
        ),
    )(e, r, s, Wa, Wb, b1, W2, b2)


def kernel(c, x, t, k_f, e_cv, e_vc, e_v_veh, cW, cb, xW, xb, tW, tb, kW, kb,
           f1W, f1b, f2W, f2b, f3W, f3b, f4W, f4b, f5W, f5b, f6W, f6b):
    emb = cW.shape[0]

    W1 = f1W.T
    W1a, W1b = W1[:emb], W1[emb:]
    W_c1 = cW.T @ W1a
    b1f = (cb @ W1a + f1b)[None, :]
    W2 = f2W.T
    b2 = f2b[None, :]

    W3 = f3W.T
    W3a, W3b = W3[:emb], W3[emb:]
    b3 = f3b[None, :]
    W4 = f4W.T
    b4 = f4b[None, :]

    W5 = f5W.T
    W5a, W5b = W5[:emb], W5[emb:]
    W_k5 = kW.T @ W5b
    b5f = (kb @ W5b + f5b)[None, :]
    W6 = f6W.T
    b6 = f6b[None, :]

    vx, vt = _embed_v(x, t, xW.T, xb[None, :], tW.T, tb[None, :], bm=1024)
    v = jnp.concatenate([vx, vt], axis=0)

    bm = 256
    cc = _stage(e_cv, v, c, W_c1, W1b, b1f, W2, b2, bm)
    vv = _stage(e_vc, cc, v, W3a, W3b, b3, W4, b4, bm)
    out = _stage(e_v_veh, vv, k_f, W_k5, W5a, b5f, W6, b6, bm)
    return out
